# Initial kernel scaffold; baseline (speedup 1.0000x reference)
import jax
import jax.numpy as jnp
from jax.experimental import pallas as pl


def kernel(x, walk_feats, hop1, hop2, hop3, segment_ids, W0_1, b0_1, W0_2, b0_2, W1_1, b1_1, W1_2, b1_2, W2_1, b2_1, W2_2, b2_2, W3_1, b3_1, W3_2, b3_2):
    raise NotImplementedError("write your pallas kernel here")


# speedup vs baseline: 1.2818x; 1.2818x over previous
import functools

import jax
import jax.numpy as jnp
from jax import lax
from jax.experimental import pallas as pl
from jax.experimental.pallas import tpu as pltpu

G = 128


def _mlp(y, w1, b1, w2, b2):
    h = jnp.maximum(jnp.dot(y, w1, preferred_element_type=jnp.float32) + b1, 0.0)
    return jnp.dot(h, w2, preferred_element_type=jnp.float32) + b2


def _body(wf_blk, x_full, h1, h2, h3, seg,
          W0_1, b0_1, W0_2, b0_2,
          W1_1, b1_1, W1_2, b1_2,
          W2_1, b2_1, W2_2, b2_2,
          W3_1, b3_1, W3_2, b3_2,
          out_ref, *, block_rows):
    i = pl.program_id(0)
    X = x_full[...]

    y = _mlp(wf_blk[...], W0_1[...], b0_1[...], W0_2[...], b0_2[...])
    y = y + _mlp(jnp.dot(h1[...], X, preferred_element_type=jnp.float32),
                 W1_1[...], b1_1[...], W1_2[...], b1_2[...])
    y = y + _mlp(jnp.dot(h2[...], X, preferred_element_type=jnp.float32),
                 W2_1[...], b2_1[...], W2_2[...], b2_2[...])
    y = y + _mlp(jnp.dot(h3[...], X, preferred_element_type=jnp.float32),
                 W3_1[...], b3_1[...], W3_2[...], b3_2[...])

    ids = jnp.broadcast_to(seg[0], (G, block_rows))
    onehotT = (lax.broadcasted_iota(jnp.int32, (G, block_rows), 0) == ids)
    contrib = jnp.dot(onehotT.astype(jnp.float32), y,
                      preferred_element_type=jnp.float32)

    @pl.when(i == 0)
    def _init():
        out_ref[...] = contrib

    @pl.when(i > 0)
    def _acc():
        out_ref[...] += contrib


def kernel(x, walk_feats, hop1, hop2, hop3, segment_ids,
           W0_1, b0_1, W0_2, b0_2,
           W1_1, b1_1, W1_2, b1_2,
           W2_1, b2_1, W2_2, b2_2,
           W3_1, b3_1, W3_2, b3_2):
    del x
    n, rw = walk_feats.shape
    h2dim = W0_2.shape[1]
    block_rows = 200
    assert n % block_rows == 0
    nblk = n // block_rows

    seg3 = segment_ids.astype(jnp.int32).reshape(nblk, 1, block_rows)
    biases = [b.reshape(1, -1) for b in (b0_1, b0_2, b1_1, b1_2,
                                         b2_1, b2_2, b3_1, b3_2)]
    (b0_1r, b0_2r, b1_1r, b1_2r, b2_1r, b2_2r, b3_1r, b3_2r) = biases

    row_spec = pl.BlockSpec((block_rows, rw), lambda i: (i, 0))
    hop_spec = pl.BlockSpec((block_rows, n), lambda i: (i, 0))
    full = lambda a: pl.BlockSpec(a.shape, lambda i: (0,) * a.ndim)

    grid_spec = pl.GridSpec(
        grid=(nblk,),
        in_specs=[
            row_spec,
            full(walk_feats),
            hop_spec, hop_spec, hop_spec,
            pl.BlockSpec((1, 1, block_rows), lambda i: (i, 0, 0)),
            full(W0_1), full(b0_1r), full(W0_2), full(b0_2r),
            full(W1_1), full(b1_1r), full(W1_2), full(b1_2r),
            full(W2_1), full(b2_1r), full(W2_2), full(b2_2r),
            full(W3_1), full(b3_1r), full(W3_2), full(b3_2r),
        ],
        out_specs=pl.BlockSpec((G, h2dim), lambda i: (0, 0)),
    )

    return pl.pallas_call(
        functools.partial(_body, block_rows=block_rows),
        grid_spec=grid_spec,
        out_shape=jax.ShapeDtypeStruct((G, h2dim), jnp.float32),
        compiler_params=pltpu.CompilerParams(
            dimension_semantics=("arbitrary",),
        ),
    )(walk_feats, walk_feats, hop1, hop2, hop3, seg3,
      W0_1, b0_1r, W0_2, b0_2r,
      W1_1, b1_1r, W1_2, b1_2r,
      W2_1, b2_1r, W2_2, b2_2r,
      W3_1, b3_1r, W3_2, b3_2r)
